# baseline (speedup 1.0000x reference)
import functools

import jax
import jax.numpy as jnp
from jax import lax
from jax.experimental import pallas as pl
from jax.experimental.pallas import tpu as pltpu
from jax.experimental.pallas import tpu_sc as plsc

_NW = 32
_CH = 16384
_NBUF = 2
_LANES = 16


def _sc_body(w_hbm, x_hbm, o_hbm, wv, in0, in1, out0, out1, in_sem, out_sem):
    c = lax.axis_index("c")
    s = lax.axis_index("s")
    wid = s * 2 + c
    per_w = x_hbm.shape[0] // _NW
    base = wid * per_w
    n_chunks = per_w // _CH

    pltpu.sync_copy(w_hbm, wv)
    f2 = wv[...] * wv[...]

    in_bufs = (in0, in1)
    out_bufs = (out0, out1)

    def in_copy(g, b):
        return pltpu.make_async_copy(
            x_hbm.at[pl.ds(base + g * _CH, _CH)], in_bufs[b], in_sem.at[b])

    def out_copy(g, b):
        return pltpu.make_async_copy(
            out_bufs[b], o_hbm.at[pl.ds(base + g * _CH, _CH)], out_sem.at[b])

    for b in range(_NBUF):
        in_copy(b, b).start()

    def outer(g2, carry):
        for b in range(_NBUF):
            g = g2 * _NBUF + b
            in_copy(g, b).wait()

            @pl.when(g2 > 0)
            def _wait_out():
                out_copy(g - _NBUF, b).wait()

            def compute(j, carry2):
                off = j * (16 * _LANES)
                for k in range(16):
                    sl = pl.ds(off + k * _LANES, _LANES)
                    out_bufs[b][sl] = in_bufs[b][sl] * f2
                return carry2

            lax.fori_loop(0, _CH // (16 * _LANES), compute, 0)
            out_copy(g, b).start()

            @pl.when(g + _NBUF < n_chunks)
            def _next_in():
                in_copy(g + _NBUF, b).start()

        return carry

    lax.fori_loop(0, n_chunks // _NBUF, outer, 0)

    for b in range(_NBUF):
        out_copy(n_chunks - _NBUF + b, b).wait()


def kernel(x, multiple_factor_weight):
    b, r, c = x.shape
    n = b * r * c
    x_flat = x.reshape(n)
    w16 = jnp.broadcast_to(multiple_factor_weight.reshape(1), (_LANES,))
    sck = functools.partial(
        pl.kernel,
        out_type=jax.ShapeDtypeStruct((n,), x.dtype),
        mesh=plsc.VectorSubcoreMesh(core_axis_name="c", subcore_axis_name="s"),
        scratch_types=[
            pltpu.VMEM((_LANES,), jnp.float32),
            pltpu.VMEM((_CH,), jnp.float32),
            pltpu.VMEM((_CH,), jnp.float32),
            pltpu.VMEM((_CH,), jnp.float32),
            pltpu.VMEM((_CH,), jnp.float32),
            pltpu.SemaphoreType.DMA((_NBUF,)),
            pltpu.SemaphoreType.DMA((_NBUF,)),
        ],
    )(_sc_body)
    out = sck(w16, x_flat)
    return out.reshape(b, r, c)


# speedup vs baseline: 3.9750x; 3.9750x over previous
import jax
import jax.numpy as jnp
from jax.experimental import pallas as pl
from jax.experimental.pallas import tpu as pltpu

_BLOCK_ROWS = 512
_NBUF = 3


def _stream_body(w_ref, x_hbm, o_hbm, in_buf, out_buf, in_sem, out_sem):
    f = w_ref[0, 0]
    f2 = f * f
    n_chunks = x_hbm.shape[0] // _BLOCK_ROWS

    def in_copy(i, slot):
        return pltpu.make_async_copy(
            x_hbm.at[pl.ds(i * _BLOCK_ROWS, _BLOCK_ROWS), :],
            in_buf.at[slot], in_sem.at[slot])

    def out_copy(i, slot):
        return pltpu.make_async_copy(
            out_buf.at[slot],
            o_hbm.at[pl.ds(i * _BLOCK_ROWS, _BLOCK_ROWS), :], out_sem.at[slot])

    for i in range(min(_NBUF, n_chunks)):
        in_copy(i, i).start()
    for i in range(n_chunks):
        slot = i % _NBUF
        in_copy(i, slot).wait()
        if i >= _NBUF:
            out_copy(i - _NBUF, slot).wait()
        out_buf[slot] = in_buf[slot] * f2
        out_copy(i, slot).start()
        if i + _NBUF < n_chunks:
            in_copy(i + _NBUF, slot).start()
    for i in range(max(0, n_chunks - _NBUF), n_chunks):
        out_copy(i, i % _NBUF).wait()


def kernel(x, multiple_factor_weight):
    b, r, c = x.shape
    n_rows = b * r
    x2d = x.reshape(n_rows, c)
    out = pl.pallas_call(
        _stream_body,
        in_specs=[
            pl.BlockSpec(memory_space=pltpu.MemorySpace.SMEM),
            pl.BlockSpec(memory_space=pltpu.MemorySpace.HBM),
        ],
        out_specs=pl.BlockSpec(memory_space=pltpu.MemorySpace.HBM),
        out_shape=jax.ShapeDtypeStruct((n_rows, c), x.dtype),
        scratch_shapes=[
            pltpu.VMEM((_NBUF, _BLOCK_ROWS, c), jnp.float32),
            pltpu.VMEM((_NBUF, _BLOCK_ROWS, c), jnp.float32),
            pltpu.SemaphoreType.DMA((_NBUF,)),
            pltpu.SemaphoreType.DMA((_NBUF,)),
        ],
    )(multiple_factor_weight, x2d)
    return out.reshape(b, r, c)


# speedup vs baseline: 3.9767x; 1.0004x over previous
import jax
import jax.numpy as jnp
from jax.experimental import pallas as pl
from jax.experimental.pallas import tpu as pltpu

_BLOCK_ROWS = 512
_NBUF = 3


def _stream_body(w_ref, x_hbm, o_hbm, in_buf, out_buf, in_sem, out_sem):
    f = w_ref[0]
    f2 = f * f
    n_chunks = x_hbm.shape[0] // _BLOCK_ROWS

    def in_copy(i, slot):
        return pltpu.make_async_copy(
            x_hbm.at[pl.ds(i * _BLOCK_ROWS, _BLOCK_ROWS), :],
            in_buf.at[slot], in_sem.at[slot])

    def out_copy(i, slot):
        return pltpu.make_async_copy(
            out_buf.at[slot],
            o_hbm.at[pl.ds(i * _BLOCK_ROWS, _BLOCK_ROWS), :], out_sem.at[slot])

    for i in range(min(_NBUF, n_chunks)):
        in_copy(i, i).start()
    for i in range(n_chunks):
        slot = i % _NBUF
        in_copy(i, slot).wait()
        if i >= _NBUF:
            out_copy(i - _NBUF, slot).wait()
        out_buf[slot] = in_buf[slot] * f2
        out_copy(i, slot).start()
        if i + _NBUF < n_chunks:
            in_copy(i + _NBUF, slot).start()
    for i in range(max(0, n_chunks - _NBUF), n_chunks):
        out_copy(i, i % _NBUF).wait()


def kernel(x, multiple_factor_weight):
    b, r, c = x.shape
    n_rows = b * r
    x2d = x.reshape(n_rows, c)
    out = pl.pallas_call(
        _stream_body,
        in_specs=[
            pl.BlockSpec(memory_space=pltpu.MemorySpace.SMEM),
            pl.BlockSpec(memory_space=pltpu.MemorySpace.HBM),
        ],
        out_specs=pl.BlockSpec(memory_space=pltpu.MemorySpace.HBM),
        out_shape=jax.ShapeDtypeStruct((n_rows, c), x.dtype),
        scratch_shapes=[
            pltpu.VMEM((_NBUF, _BLOCK_ROWS, c), jnp.float32),
            pltpu.VMEM((_NBUF, _BLOCK_ROWS, c), jnp.float32),
            pltpu.SemaphoreType.DMA((_NBUF,)),
            pltpu.SemaphoreType.DMA((_NBUF,)),
        ],
    )(multiple_factor_weight.reshape(1), x2d)
    return out.reshape(b, r, c)


# speedup vs baseline: 3.9780x; 1.0003x over previous
import jax
import jax.numpy as jnp
from jax.experimental import pallas as pl
from jax.experimental.pallas import tpu as pltpu

_BLOCK_ROWS = 512
_NBUF = 2


def _stream_body(w_ref, x_hbm, o_hbm, in_buf, out_buf, in_sem, out_sem):
    f = w_ref[0]
    f2 = f * f
    n_chunks = x_hbm.shape[0] // _BLOCK_ROWS

    def in_copy(i, slot):
        return pltpu.make_async_copy(
            x_hbm.at[pl.ds(i * _BLOCK_ROWS, _BLOCK_ROWS), :],
            in_buf.at[slot], in_sem.at[slot])

    def out_copy(i, slot):
        return pltpu.make_async_copy(
            out_buf.at[slot],
            o_hbm.at[pl.ds(i * _BLOCK_ROWS, _BLOCK_ROWS), :], out_sem.at[slot])

    for i in range(min(_NBUF, n_chunks)):
        in_copy(i, i).start()
    for i in range(n_chunks):
        slot = i % _NBUF
        in_copy(i, slot).wait()
        if i >= _NBUF:
            out_copy(i - _NBUF, slot).wait()
        out_buf[slot] = in_buf[slot] * f2
        out_copy(i, slot).start()
        if i + _NBUF < n_chunks:
            in_copy(i + _NBUF, slot).start()
    for i in range(max(0, n_chunks - _NBUF), n_chunks):
        out_copy(i, i % _NBUF).wait()


def kernel(x, multiple_factor_weight):
    b, r, c = x.shape
    n_rows = b * r
    x2d = x.reshape(n_rows, c)
    out = pl.pallas_call(
        _stream_body,
        in_specs=[
            pl.BlockSpec(memory_space=pltpu.MemorySpace.SMEM),
            pl.BlockSpec(memory_space=pltpu.MemorySpace.HBM),
        ],
        out_specs=pl.BlockSpec(memory_space=pltpu.MemorySpace.HBM),
        out_shape=jax.ShapeDtypeStruct((n_rows, c), x.dtype),
        compiler_params=pltpu.CompilerParams(
            skip_device_barrier=True,
            disable_bounds_checks=True,
        ),
        scratch_shapes=[
            pltpu.VMEM((_NBUF, _BLOCK_ROWS, c), jnp.float32),
            pltpu.VMEM((_NBUF, _BLOCK_ROWS, c), jnp.float32),
            pltpu.SemaphoreType.DMA((_NBUF,)),
            pltpu.SemaphoreType.DMA((_NBUF,)),
        ],
    )(multiple_factor_weight.reshape(1), x2d)
    return out.reshape(b, r, c)


# speedup vs baseline: 3.9790x; 1.0003x over previous
import jax
import jax.numpy as jnp
from jax.experimental import pallas as pl
from jax.experimental.pallas import tpu as pltpu

_BLOCK_ROWS = 512
_NBUF = 3


def _stream_body(w_ref, x_hbm, o_hbm, in_buf, out_buf, in_sem, out_sem):
    f = w_ref[0]
    f2 = f * f
    n_chunks = x_hbm.shape[0] // _BLOCK_ROWS

    def in_copy(i, slot):
        return pltpu.make_async_copy(
            x_hbm.at[pl.ds(i * _BLOCK_ROWS, _BLOCK_ROWS), :],
            in_buf.at[slot], in_sem.at[slot])

    def out_copy(i, slot):
        return pltpu.make_async_copy(
            out_buf.at[slot],
            o_hbm.at[pl.ds(i * _BLOCK_ROWS, _BLOCK_ROWS), :], out_sem.at[slot])

    for i in range(min(_NBUF, n_chunks)):
        in_copy(i, i).start()
    for i in range(n_chunks):
        slot = i % _NBUF
        in_copy(i, slot).wait()
        if i >= _NBUF:
            out_copy(i - _NBUF, slot).wait()
        out_buf[slot] = in_buf[slot] * f2
        out_copy(i, slot).start()
        if i + _NBUF < n_chunks:
            in_copy(i + _NBUF, slot).start()
    for i in range(max(0, n_chunks - _NBUF), n_chunks):
        out_copy(i, i % _NBUF).wait()


def kernel(x, multiple_factor_weight):
    b, r, c = x.shape
    n_rows = b * r
    x2d = x.reshape(n_rows, c)
    out = pl.pallas_call(
        _stream_body,
        in_specs=[
            pl.BlockSpec(memory_space=pltpu.MemorySpace.SMEM),
            pl.BlockSpec(memory_space=pltpu.MemorySpace.HBM),
        ],
        out_specs=pl.BlockSpec(memory_space=pltpu.MemorySpace.HBM),
        out_shape=jax.ShapeDtypeStruct((n_rows, c), x.dtype),
        compiler_params=pltpu.CompilerParams(
            skip_device_barrier=True,
            disable_bounds_checks=True,
        ),
        scratch_shapes=[
            pltpu.VMEM((_NBUF, _BLOCK_ROWS, c), jnp.float32),
            pltpu.VMEM((_NBUF, _BLOCK_ROWS, c), jnp.float32),
            pltpu.SemaphoreType.DMA((_NBUF,)),
            pltpu.SemaphoreType.DMA((_NBUF,)),
        ],
    )(multiple_factor_weight.reshape(1), x2d)
    return out.reshape(b, r, c)
